# Initial kernel scaffold; baseline (speedup 1.0000x reference)
import jax
import jax.numpy as jnp
from jax.experimental import pallas as pl


def kernel(nodes, edges, senders, receivers, W_msg, b_msg, W_n1, b_n1, W_n2, b_n2):
    raise NotImplementedError("write your pallas kernel here")


# speedup vs baseline: 2.9908x; 2.9908x over previous
import functools

import jax
import jax.numpy as jnp
from jax import lax
from jax.experimental import pallas as pl
from jax.experimental.pallas import tpu as pltpu
from jax.experimental.pallas import tpu_sc as plsc

N_NODES = 10000
N_EDGES = 320000
D = 128

NC = 2
NS = 16
NW = NC * NS
LANES = 16

EDGES_PER_W = N_EDGES // NW
CHUNK = 80
NCHUNKS = EDGES_PER_W // CHUNK
N_PAD = 10240
ROWS_PER_S = N_PAD // NS
ZROWS = 128


def _psr_body(nodes_ref, w_ref, b_ref, ps_ref, pr_ref):
    n = nodes_ref[...]
    w = w_ref[...]
    ps_ref[...] = (
        jnp.dot(n, w[0:D, :], preferred_element_type=jnp.float32) + b_ref[...]
    )
    pr_ref[...] = jnp.dot(n, w[D : 2 * D, :], preferred_element_type=jnp.float32)


def _e_body(edges_ref, w_ref, e_ref):
    e_ref[...] = jnp.dot(
        edges_ref[...], w_ref[2 * D :, :], preferred_element_type=jnp.float32
    )


def _final_body(nodes_ref, agg_ref, w1_ref, b1_ref, w2_ref, b2_ref, out_ref):
    n = nodes_ref[...]
    agg = agg_ref[0] + agg_ref[1]
    h = jnp.dot(n, w1_ref[0:D, :], preferred_element_type=jnp.float32)
    h = h + jnp.dot(agg, w1_ref[D:, :], preferred_element_type=jnp.float32)
    h = jnp.maximum(h + b1_ref[...], 0.0)
    h = jnp.dot(h, w2_ref[...], preferred_element_type=jnp.float32) + b2_ref[...]
    out_ref[...] = n + h


def _sc_body(
    ps_hbm, pr_hbm, e_hbm, snd_hbm, rcv_hbm, out_hbm,
    agg_sh, zb, sidx, ridx, ps_v, pr_v, e_v, sem,
):
    c = lax.axis_index("c")
    s = lax.axis_index("s")

    def _zrow(i, _):
        for g in range(D // LANES):
            zb[i, pl.ds(g * LANES, LANES)] = jnp.zeros((LANES,), jnp.float32)
        return 0

    lax.fori_loop(0, ZROWS, _zrow, 0)
    for j in range(ROWS_PER_S // ZROWS):
        pltpu.sync_copy(zb, agg_sh.at[pl.ds(s * ROWS_PER_S + j * ZROWS, ZROWS)])
    plsc.subcore_barrier()

    w = s * NC + c
    base_w = w * EDGES_PER_W

    def _chunk(j, _):
        base = pl.multiple_of(base_w + j * CHUNK, 16)
        pltpu.sync_copy(snd_hbm.at[pl.ds(base, CHUNK)], sidx)
        pltpu.sync_copy(rcv_hbm.at[pl.ds(base, CHUNK)], ridx)
        pltpu.sync_copy(e_hbm.at[pl.ds(base, CHUNK)], e_v)
        pltpu.async_copy(ps_hbm.at[sidx], ps_v, sem).wait()
        pltpu.async_copy(pr_hbm.at[ridx], pr_v, sem).wait()

        def _row(i, _):
            for g in range(D // LANES):
                sl = pl.ds(g * LANES, LANES)
                v = ps_v[i, sl] + pr_v[i, sl] + e_v[i, sl]
                e_v[i, sl] = jnp.maximum(v, 0.0)
            return 0

        lax.fori_loop(0, CHUNK, _row, 0)
        pltpu.sync_copy(e_v, agg_sh.at[ridx], add=True)
        return 0

    lax.fori_loop(0, NCHUNKS, _chunk, 0)
    plsc.subcore_barrier()

    for j in range(ROWS_PER_S // ZROWS):
        row0 = s * ROWS_PER_S + j * ZROWS
        pltpu.sync_copy(agg_sh.at[pl.ds(row0, ZROWS)], zb)
        pltpu.sync_copy(zb, out_hbm.at[c, pl.ds(row0, ZROWS)])


def _segment_messages(ps, pr, e, senders, receivers):
    mesh = plsc.VectorSubcoreMesh(
        core_axis_name="c", subcore_axis_name="s", num_cores=NC, num_subcores=NS
    )
    return pl.kernel(
        _sc_body,
        out_type=jax.ShapeDtypeStruct((NC, N_PAD, D), jnp.float32),
        mesh=mesh,
        scratch_types=[
            pltpu.VMEM_SHARED((N_PAD, D), jnp.float32),
            pltpu.VMEM((ZROWS, D), jnp.float32),
            pltpu.VMEM((CHUNK,), jnp.int32),
            pltpu.VMEM((CHUNK,), jnp.int32),
            pltpu.VMEM((CHUNK, D), jnp.float32),
            pltpu.VMEM((CHUNK, D), jnp.float32),
            pltpu.VMEM((CHUNK, D), jnp.float32),
            pltpu.SemaphoreType.DMA,
        ],
    )(ps, pr, e, senders, receivers)


def kernel(nodes, edges, senders, receivers, W_msg, b_msg, W_n1, b_n1, W_n2, b_n2):
    b_msg2 = b_msg.reshape(1, D)
    b1 = b_n1.reshape(1, D)
    b2 = b_n2.reshape(1, D)

    ps, pr = pl.pallas_call(
        _psr_body,
        out_shape=(
            jax.ShapeDtypeStruct((N_NODES, D), jnp.float32),
            jax.ShapeDtypeStruct((N_NODES, D), jnp.float32),
        ),
    )(nodes, W_msg, b_msg2)

    eblk = 8000
    e = pl.pallas_call(
        _e_body,
        grid=(N_EDGES // eblk,),
        in_specs=[
            pl.BlockSpec((eblk, 16), lambda i: (i, 0)),
            pl.BlockSpec((2 * D + 16, D), lambda i: (0, 0)),
        ],
        out_specs=pl.BlockSpec((eblk, D), lambda i: (i, 0)),
        out_shape=jax.ShapeDtypeStruct((N_EDGES, D), jnp.float32),
    )(edges, W_msg)

    agg2 = _segment_messages(ps, pr, e, senders, receivers)[:, :N_NODES, :]

    nblk = 1000
    out = pl.pallas_call(
        _final_body,
        grid=(N_NODES // nblk,),
        in_specs=[
            pl.BlockSpec((nblk, D), lambda i: (i, 0)),
            pl.BlockSpec((NC, nblk, D), lambda i: (0, i, 0)),
            pl.BlockSpec((2 * D, D), lambda i: (0, 0)),
            pl.BlockSpec((1, D), lambda i: (0, 0)),
            pl.BlockSpec((D, D), lambda i: (0, 0)),
            pl.BlockSpec((1, D), lambda i: (0, 0)),
        ],
        out_specs=pl.BlockSpec((nblk, D), lambda i: (i, 0)),
        out_shape=jax.ShapeDtypeStruct((N_NODES, D), jnp.float32),
    )(nodes, agg2, W_n1, b1, W_n2, b2)
    return out
